# Initial kernel scaffold; baseline (speedup 1.0000x reference)
import jax
import jax.numpy as jnp
from jax.experimental import pallas as pl


def kernel(bs_antenna_indices, ue_antenna_indices, embeddings):
    raise NotImplementedError("write your pallas kernel here")


# speedup vs baseline: 2.0058x; 2.0058x over previous
import functools

import jax
import jax.numpy as jnp
from jax import lax
from jax.experimental import pallas as pl
from jax.experimental.pallas import tpu as pltpu
from jax.experimental.pallas import tpu_sc as plsc

_NUM_BS = 256
_NUM_UE = 8
_EMB_DIM = 64
_BATCH = 16384

_INFO = plsc.get_sparse_core_info()
_NC = _INFO.num_cores
_NS = _INFO.num_subcores
_L = _INFO.num_lanes
_NW = _NC * _NS
_BPW = _BATCH // _NW
_CHUNK = 128
_NCHUNK = _BPW // _CHUNK

_mesh = plsc.VectorSubcoreMesh(core_axis_name="c", subcore_axis_name="s")


@functools.partial(
    pl.kernel,
    out_type=jax.ShapeDtypeStruct((_BATCH, _EMB_DIM), jnp.float32),
    mesh=_mesh,
    scratch_types=[
        pltpu.VMEM((_BPW,), jnp.int32),
        pltpu.VMEM((_BPW,), jnp.int32),
        pltpu.VMEM((_NCHUNK, _CHUNK), jnp.int32),
        pltpu.VMEM((_BPW, _EMB_DIM), jnp.float32),
        pltpu.SemaphoreType.DMA,
    ],
    compiler_params=pltpu.CompilerParams(use_tc_tiling_on_sc=False),
)
def _gather_kernel(bs_hbm, ue_hbm, tab_hbm, out_hbm,
                   bs_v, ue_v, idx_v, rows_v, sem):
    wid = lax.axis_index("s") * _NC + lax.axis_index("c")
    base = wid * _BPW
    pltpu.sync_copy(bs_hbm.at[pl.ds(base, _BPW)], bs_v)
    pltpu.sync_copy(ue_hbm.at[pl.ds(base, _BPW)], ue_v)
    for i in range(_BPW // _L):
        b = bs_v[pl.ds(i * _L, _L)]
        u = ue_v[pl.ds(i * _L, _L)]
        idx_v[i // (_CHUNK // _L), pl.ds((i % (_CHUNK // _L)) * _L, _L)] = (
            b * _NUM_UE + u)
    copies = [
        pltpu.async_copy(tab_hbm.at[idx_v.at[j]],
                         rows_v.at[pl.ds(j * _CHUNK, _CHUNK)], sem)
        for j in range(_NCHUNK)
    ]
    for c in copies:
        c.wait()
    pltpu.sync_copy(rows_v, out_hbm.at[pl.ds(base, _BPW)])


def kernel(bs_antenna_indices, ue_antenna_indices, embeddings):
    flat_table = embeddings.reshape(_NUM_BS * _NUM_UE, _EMB_DIM)
    return _gather_kernel(bs_antenna_indices.astype(jnp.int32),
                          ue_antenna_indices.astype(jnp.int32),
                          flat_table)
